# baseline (speedup 1.0000x reference)
import jax
import jax.numpy as jnp
from jax import lax
from jax.experimental import pallas as pl
from jax.experimental.pallas import tpu as pltpu
from jax.experimental.pallas import tpu_sc as plsc

_NC = 2
_NS = 16
_NW = _NC * _NS
_CH = 100
_GB = 2
_NBUF = 2
_D = 64


def _fused_table_body(t_ref, wt_ref, b_ref, o_ref):
    o_ref[...] = (
        jnp.dot(t_ref[...], wt_ref[...], preferred_element_type=jnp.float32)
        + b_ref[...]
    )


def _gather_body(fused_hbm, idx_hbm, out_hbm, fused_s, idx_v, big,
                 gsem0, gsem1, ssem0, ssem1):
    gsem = (gsem0, gsem1)
    ssem = (ssem0, ssem1)
    wid = lax.axis_index("s") * _NC + lax.axis_index("c")
    nch = idx_hbm.shape[0] // _NW
    nbatch = nch * _CH // 200
    ngroups = nbatch // _GB
    batch0 = wid * nbatch
    cpg = _GB * 200 // _CH

    @pl.when(lax.axis_index("s") == 0)
    def _():
        pltpu.sync_copy(fused_hbm, fused_s)

    plsc.subcore_barrier()
    pltpu.sync_copy(idx_hbm.at[pl.ds(wid * nch, nch)], idx_v)

    def gather_copy(g, b, k):
        return pltpu.make_async_copy(
            fused_s.at[idx_v.at[g * cpg + k]],
            big.at[b, k // 2, pl.ds((k % 2) * _CH, _CH)],
            gsem[b],
        )

    def scatter_copy(g, b):
        return pltpu.make_async_copy(
            big.at[b],
            out_hbm.at[pl.ds(batch0 + g * _GB, _GB)],
            ssem[b],
        )

    def fire_gathers(g, b):
        for k in range(cpg):
            gather_copy(g, b, k).start()

    fire_gathers(0, 0)

    def step(i, carry):
        for b in range(_NBUF):
            g = i * _NBUF + b
            for k in range(cpg):
                gather_copy(g, b, k).wait()
            scatter_copy(g, b).start()
            b2 = (b + 1) % _NBUF

            @pl.when(g + 1 < ngroups)
            def _():
                @pl.when(g >= 1)
                def _():
                    scatter_copy(g - 1, b2).wait()

                fire_gathers(g + 1, b2)

        return carry

    lax.fori_loop(0, ngroups // _NBUF, step, 0)
    scatter_copy(ngroups - 2, (ngroups - 2) % _NBUF).wait()
    scatter_copy(ngroups - 1, (ngroups - 1) % _NBUF).wait()


def kernel(src, table, W, b):
    B, S = src.shape
    total = B * S
    tpad = jnp.pad(table, ((0, (-table.shape[0]) % 8), (0, 0)))
    fused = pl.pallas_call(
        _fused_table_body,
        out_shape=jax.ShapeDtypeStruct((tpad.shape[0], _D), jnp.float32),
    )(tpad, W.T, b[None, :])

    idx = src.reshape(total // _CH, _CH)
    nch = idx.shape[0] // _NW
    out = pl.kernel(
        _gather_body,
        out_type=jax.ShapeDtypeStruct((B, S, _D), jnp.float32),
        mesh=plsc.VectorSubcoreMesh(core_axis_name="c", subcore_axis_name="s"),
        compiler_params=pltpu.CompilerParams(use_tc_tiling_on_sc=False),
        scratch_types=[
            pltpu.VMEM_SHARED(fused.shape, jnp.float32),
            pltpu.VMEM((nch, _CH), jnp.int32),
            pltpu.VMEM((_NBUF, _GB, S, _D), jnp.float32),
            pltpu.SemaphoreType.DMA,
            pltpu.SemaphoreType.DMA,
            pltpu.SemaphoreType.DMA,
            pltpu.SemaphoreType.DMA,
        ],
    )(fused, idx)
    return out
